# baseline (speedup 1.0000x reference)
import functools

import jax
import jax.numpy as jnp
from jax import lax
from jax.experimental import pallas as pl
from jax.experimental.pallas import tpu as pltpu
from jax.experimental.pallas import tpu_sc as plsc


def _make_gather(total: int, D: int):
    info = plsc.get_sparse_core_info()
    NC, NS = info.num_cores, info.num_subcores
    NW = NC * NS
    assert total % NW == 0
    b_per_w = total // NW
    C = 800
    assert b_per_w % C == 0
    nchunk = b_per_w // C

    mesh = plsc.VectorSubcoreMesh(core_axis_name="c", subcore_axis_name="s")

    @functools.partial(
        pl.kernel,
        mesh=mesh,
        out_type=jax.ShapeDtypeStruct((total, D), jnp.float32),
        scratch_types=[
            pltpu.VMEM((b_per_w,), jnp.int32),
            pltpu.VMEM((2, C, D), jnp.float32),
            pltpu.SemaphoreType.DMA,
            pltpu.SemaphoreType.DMA,
        ],
        compiler_params=pltpu.CompilerParams(use_tc_tiling_on_sc=False),
    )
    def gather_kernel(table_hbm, idx_hbm, out_hbm, idx_v, rows_v, gsem0, gsem1):
        wid = lax.axis_index("s") * NC + lax.axis_index("c")
        base = wid * b_per_w
        pltpu.sync_copy(idx_hbm.at[pl.ds(base, b_per_w)], idx_v)
        gsems = (gsem0, gsem1)
        cp0 = pltpu.async_copy(
            table_hbm.at[idx_v.at[pl.ds(0, C)]], rows_v.at[0], gsems[0])
        copies = [cp0, None]
        for c in range(nchunk):
            buf = c % 2
            if c + 1 < nchunk:
                nbuf = (c + 1) % 2
                copies[nbuf] = pltpu.async_copy(
                    table_hbm.at[idx_v.at[pl.ds((c + 1) * C, C)]],
                    rows_v.at[nbuf], gsems[nbuf])
            copies[buf].wait()
            pltpu.sync_copy(rows_v.at[buf], out_hbm.at[pl.ds(base + c * C, C)])

    return gather_kernel


def kernel(item_ids, item_codes):
    B, H = item_ids.shape
    N, D = item_codes.shape
    total = B * H
    flat_ids = item_ids.reshape(total).astype(jnp.int32)
    out = _make_gather(total, D)(item_codes, flat_ids)
    return out.reshape(B, H, D)
